# baseline (speedup 1.0000x reference)
import math

import jax
import jax.numpy as jnp
from jax import lax
from jax.experimental import pallas as pl
from jax.experimental.pallas import tpu as pltpu

_N = 4096
_V = 32000
_SMOOTHING = 0.1
_BR = 256

_S = _SMOOTHING / (_V - 3)
_CONF = 1.0 - _SMOOTHING
_C2 = (_V - 2) * _S * math.log(_S)
_C3 = (_V - 3) * _S * math.log(_S) + _CONF * math.log(_CONF)


def _loss_block(ts_ref, x_ref, t_ref, out_ref, win_ref):
    i = pl.program_id(0)
    x = x_ref[...]
    t = t_ref[0]

    rs = jnp.sum(x, axis=1, keepdims=True)
    s_i = rs - x[:, 0:1] - x[:, 1:2]
    reg = t >= 2
    dense = jnp.sum(jnp.where(t != 0, s_i, 0.0))
    n_reg = jnp.sum(reg.astype(jnp.float32))
    n_one = jnp.sum((t == 1).astype(jnp.float32))

    for r in range(_BR):
        c0 = pl.multiple_of((ts_ref[0, 0, r] // 128) * 128, 128)
        win_ref[pl.ds(r, 1), :] = x_ref[pl.ds(r, 1), pl.ds(c0, 128)]
    lane = t % 128
    col = lax.broadcasted_iota(jnp.int32, (_BR, 128), 1)
    gath = jnp.sum(jnp.where((col == lane) & reg, win_ref[...], 0.0))

    partial = jnp.reshape(_C3 * n_reg + _C2 * n_one
                          - _S * dense - (_CONF - _S) * gath, (1, 1))

    @pl.when(i == 0)
    def _init():
        out_ref[...] = partial

    @pl.when(i != 0)
    def _acc():
        out_ref[...] += partial


def kernel(model_output_dist, target_sequence):
    n, v = model_output_dist.shape
    nb = n // _BR
    t = target_sequence.astype(jnp.int32)
    out = pl.pallas_call(
        _loss_block,
        grid=(nb,),
        in_specs=[
            pl.BlockSpec((1, 1, _BR), lambda i: (i, 0, 0),
                         memory_space=pltpu.SMEM),
            pl.BlockSpec((_BR, v), lambda i: (i, 0)),
            pl.BlockSpec((1, _BR, 1), lambda i: (i, 0, 0)),
        ],
        out_specs=pl.BlockSpec((1, 1), lambda i: (0, 0)),
        out_shape=jax.ShapeDtypeStruct((1, 1), jnp.float32),
        scratch_shapes=[pltpu.VMEM((_BR, 128), jnp.float32)],
        compiler_params=pltpu.CompilerParams(
            vmem_limit_bytes=112 * 1024 * 1024),
    )(t.reshape(nb, 1, _BR), model_output_dist, t.reshape(nb, _BR, 1))
    return out[0, 0]


# speedup vs baseline: 1.0165x; 1.0165x over previous
import math

import jax
import jax.numpy as jnp
from jax import lax
from jax.experimental import pallas as pl
from jax.experimental.pallas import tpu as pltpu

_N = 4096
_V = 32000
_SMOOTHING = 0.1
_BR = 128

_S = _SMOOTHING / (_V - 3)
_CONF = 1.0 - _SMOOTHING
_C2 = (_V - 2) * _S * math.log(_S)
_C3 = (_V - 3) * _S * math.log(_S) + _CONF * math.log(_CONF)


def _loss_block(ts_ref, x_ref, t_ref, out_ref, win_ref):
    i = pl.program_id(0)
    x = x_ref[...]
    t = t_ref[0]

    rs = jnp.sum(x, axis=1, keepdims=True)
    s_i = rs - x[:, 0:1] - x[:, 1:2]
    reg = t >= 2
    dense = jnp.sum(jnp.where(t != 0, s_i, 0.0))
    n_reg = jnp.sum(reg.astype(jnp.float32))
    n_one = jnp.sum((t == 1).astype(jnp.float32))

    for r in range(_BR):
        c0 = pl.multiple_of((ts_ref[0, 0, r] // 128) * 128, 128)
        win_ref[pl.ds(r, 1), :] = x_ref[pl.ds(r, 1), pl.ds(c0, 128)]
    lane = t % 128
    col = lax.broadcasted_iota(jnp.int32, (_BR, 128), 1)
    gath = jnp.sum(jnp.where((col == lane) & reg, win_ref[...], 0.0))

    partial = jnp.reshape(_C3 * n_reg + _C2 * n_one
                          - _S * dense - (_CONF - _S) * gath, (1, 1))

    @pl.when(i == 0)
    def _init():
        out_ref[...] = partial

    @pl.when(i != 0)
    def _acc():
        out_ref[...] += partial


def kernel(model_output_dist, target_sequence):
    n, v = model_output_dist.shape
    nb = n // _BR
    t = target_sequence.astype(jnp.int32)
    out = pl.pallas_call(
        _loss_block,
        grid=(nb,),
        in_specs=[
            pl.BlockSpec((1, 1, _BR), lambda i: (i, 0, 0),
                         memory_space=pltpu.SMEM),
            pl.BlockSpec((_BR, v), lambda i: (i, 0)),
            pl.BlockSpec((1, _BR, 1), lambda i: (i, 0, 0)),
        ],
        out_specs=pl.BlockSpec((1, 1), lambda i: (0, 0)),
        out_shape=jax.ShapeDtypeStruct((1, 1), jnp.float32),
        scratch_shapes=[pltpu.VMEM((_BR, 128), jnp.float32)],
    )(t.reshape(nb, 1, _BR), model_output_dist, t.reshape(nb, _BR, 1))
    return out[0, 0]
